# baseline (speedup 1.0000x reference)
import numpy as np
import jax
import jax.numpy as jnp
from jax.experimental import pallas as pl
from jax.experimental.pallas import tpu as pltpu

_N = 8192
_M = 3276
_M4 = 3328
_S = 512
_K = 32
_TQ = 128
_R2 = np.float32(0.2 * 0.2)
_BIG = np.float32(1e9)
_BIGN = np.float32(_N)


def _select_kernel(ncol_ref, nrow_ref, x4_ref, out_ref):
    n_col = ncol_ref[0]
    ii = jax.lax.broadcasted_iota(jnp.int32, (_N, 1), 0).astype(jnp.float32)
    TJ = 128
    jj0 = jax.lax.broadcasted_iota(jnp.int32, (1, TJ), 1).astype(jnp.float32)

    def rank_body(jt, rank):
        nj = nrow_ref[0, :, pl.ds(jt * TJ, TJ)]
        jj = jj0 + jnp.float32(jt) * np.float32(TJ)
        less = (nj < n_col) | ((nj == n_col) & (jj < ii))
        return rank + jnp.sum(less.astype(jnp.float32), axis=1, keepdims=True)

    rank = jax.lax.fori_loop(0, _N // TJ, rank_body,
                             jnp.zeros((_N, 1), jnp.float32))
    TR = 128
    rr0 = jax.lax.broadcasted_iota(jnp.int32, (1, TR), 1).astype(jnp.float32)

    def scatter_body(rt, _):
        rr = rr0 + jnp.float32(rt) * np.float32(TR)
        e = (rank == rr).astype(jnp.float32)
        for d in range(4):
            col = x4_ref[0, :, d:d + 1]
            out_ref[0, d:d + 1, pl.ds(rt * TR, TR)] = jnp.sum(
                e * col, axis=0, keepdims=True)
        return 0

    jax.lax.fori_loop(0, _M4 // TR, scatter_body, 0)


def _fps_kernel(xc_ref, curv_ref, ox_ref, oy_ref, oz_ref, oc_ref):
    xc = xc_ref[...]
    x0 = xc[:, 0, :]
    x1 = xc[:, 1, :]
    x2 = xc[:, 2, :]
    x3 = xc[:, 3, :]
    curv = curv_ref[...]
    Bb = x0.shape[0]
    li = jax.lax.broadcasted_iota(jnp.int32, (Bb, _M4), 1).astype(jnp.float32)
    mask = li < np.float32(_M)
    li_n = jax.lax.broadcasted_iota(jnp.int32, (Bb, _N), 1).astype(jnp.float32)

    li_s = jax.lax.broadcasted_iota(jnp.int32, (Bb, _S), 1)

    sel0 = li_s == 0
    z = jnp.zeros((Bb, _S), jnp.float32)
    a0 = jnp.where(sel0, x0[:, 0:1], z)
    a1 = jnp.where(sel0, x1[:, 0:1], z)
    a2 = jnp.where(sel0, x2[:, 0:1], z)
    ac = jnp.where(sel0, curv[:, 0:1], z)
    dists0 = jnp.full((Bb, _M4), 1e10, jnp.float32)

    def body(t, carry):
        dists, l0, l1, l2, l3, a0, a1, a2, ac = carry
        d = ((((x0 - l0) * (x0 - l0)) + ((x1 - l1) * (x1 - l1)))
             + ((x2 - l2) * (x2 - l2))) + ((x3 - l3) * (x3 - l3))
        dists = jnp.where(mask, jnp.minimum(dists, d), -1.0)
        m = jnp.max(dists, axis=1, keepdims=True)
        nxt = jnp.min(jnp.where(dists == m, li, _BIG), axis=1, keepdims=True)
        oh = (li == nxt).astype(jnp.float32)
        g0 = jnp.sum(x0 * oh, axis=1, keepdims=True)
        g1 = jnp.sum(x1 * oh, axis=1, keepdims=True)
        g2 = jnp.sum(x2 * oh, axis=1, keepdims=True)
        g3 = jnp.sum(x3 * oh, axis=1, keepdims=True)
        gc = jnp.sum(curv * (li_n == nxt).astype(jnp.float32), axis=1, keepdims=True)
        sel = li_s == t
        a0 = jnp.where(sel, g0, a0)
        a1 = jnp.where(sel, g1, a1)
        a2 = jnp.where(sel, g2, a2)
        ac = jnp.where(sel, gc, ac)
        return (dists, g0, g1, g2, g3, a0, a1, a2, ac)

    carry0 = (dists0, x0[:, 0:1], x1[:, 0:1], x2[:, 0:1], x3[:, 0:1],
              a0, a1, a2, ac)
    _, _, _, _, _, a0, a1, a2, ac = jax.lax.fori_loop(1, _S, body, carry0)
    ox_ref[...] = a0
    oy_ref[...] = a1
    oz_ref[...] = a2
    oc_ref[...] = ac


def _group_mlp_kernel(q_ref, xt_ref, x3_ref, f_ref, w1_ref, b1_ref, w2_ref,
                      b2_ref, out_ref, masked_ref, first_ref, pooled_ref):
    q = q_ref[0]
    xt = xt_ref[0]
    x3 = x3_ref[0]
    fb = f_ref[0]
    qn = (q[:, 0:1] * q[:, 0:1] + q[:, 1:2] * q[:, 1:2]) + q[:, 2:3] * q[:, 2:3]
    xn = (xt[0:1, :] * xt[0:1, :] + xt[1:2, :] * xt[1:2, :]) + xt[2:3, :] * xt[2:3, :]
    cross = jnp.dot(q, xt, preferred_element_type=jnp.float32)
    d2 = (qn + xn) - 2.0 * cross
    li = jax.lax.broadcasted_iota(jnp.int32, (_TQ, _N), 1).astype(jnp.float32)
    masked_ref[...] = jnp.where(d2 > _R2, _BIGN, li)
    pooled_ref[...] = jnp.zeros((_TQ, 128), jnp.float32)
    first_ref[...] = jnp.zeros((_TQ, 1), jnp.float32)

    def body(k, _):
        msk = masked_ref[...]
        cur = jnp.min(msk, axis=1, keepdims=True)
        first = jnp.where(k == 0, cur, first_ref[...])
        first_ref[...] = first
        sel = jnp.where(cur >= _BIGN, first, cur)
        sel = jnp.minimum(sel, np.float32(_N - 1))
        oh = (li == sel).astype(jnp.float32)
        masked_ref[...] = jnp.where(msk == cur, _BIGN, msk)
        g3 = jax.lax.dot_general(oh, x3, (((1,), (0,)), ((), ())),
                                 precision=jax.lax.Precision.HIGHEST,
                                 preferred_element_type=jnp.float32)
        gf = jax.lax.dot_general(oh.astype(jnp.bfloat16), fb,
                                 (((1,), (0,)), ((), ())),
                                 preferred_element_type=jnp.float32)
        h = jnp.concatenate([g3 - q, gf], axis=1)
        l1 = jnp.maximum(jnp.dot(h, w1_ref[...], preferred_element_type=jnp.float32)
                         + b1_ref[...], 0.0)
        l2 = jnp.maximum(jnp.dot(l1, w2_ref[...], preferred_element_type=jnp.float32)
                         + b2_ref[...], 0.0)
        pooled_ref[...] = jnp.maximum(pooled_ref[...], l2)
        return 0

    jax.lax.fori_loop(0, _K, body, 0)
    out_ref[0] = pooled_ref[...]


def kernel(xyz, features, curvature, W1, b1, W2, b2):
    Bb, Nn, _ = xyz.shape
    C = features.shape[-1]

    x4 = jnp.concatenate([xyz, curvature[:, :, None]], axis=2)
    norms = jnp.linalg.norm(xyz, axis=2)

    xc = pl.pallas_call(
        _select_kernel,
        grid=(Bb,),
        in_specs=[
            pl.BlockSpec((1, _N, 1), lambda b: (b, 0, 0)),
            pl.BlockSpec((1, 1, _N), lambda b: (b, 0, 0)),
            pl.BlockSpec((1, _N, 4), lambda b: (b, 0, 0)),
        ],
        out_specs=pl.BlockSpec((1, 4, _M4), lambda b: (b, 0, 0)),
        out_shape=jax.ShapeDtypeStruct((Bb, 4, _M4), jnp.float32),
    )(norms[:, :, None], norms[:, None, :], x4)

    ox, oy, oz, oc = pl.pallas_call(
        _fps_kernel,
        out_shape=[jax.ShapeDtypeStruct((Bb, _S), jnp.float32)] * 4,
    )(xc, curvature)

    new_xyz = jnp.stack([ox, oy, oz], axis=-1)
    new_curvature = oc

    feat_bf = features.astype(jnp.bfloat16)
    xyzt = jnp.transpose(xyz, (0, 2, 1))

    pooled = pl.pallas_call(
        _group_mlp_kernel,
        grid=(Bb, _S // _TQ),
        in_specs=[
            pl.BlockSpec((1, _TQ, 3), lambda b, s: (b, s, 0)),
            pl.BlockSpec((1, 3, _N), lambda b, s: (b, 0, 0)),
            pl.BlockSpec((1, _N, 3), lambda b, s: (b, 0, 0)),
            pl.BlockSpec((1, _N, C), lambda b, s: (b, 0, 0)),
            pl.BlockSpec(W1.T.shape, lambda b, s: (0, 0)),
            pl.BlockSpec((1, 64), lambda b, s: (0, 0)),
            pl.BlockSpec(W2.T.shape, lambda b, s: (0, 0)),
            pl.BlockSpec((1, 128), lambda b, s: (0, 0)),
        ],
        out_specs=pl.BlockSpec((1, _TQ, 128), lambda b, s: (b, s, 0)),
        out_shape=jax.ShapeDtypeStruct((Bb, _S, 128), jnp.float32),
        scratch_shapes=[
            pltpu.VMEM((_TQ, _N), jnp.float32),
            pltpu.VMEM((_TQ, 1), jnp.float32),
            pltpu.VMEM((_TQ, 128), jnp.float32),
        ],
    )(new_xyz, xyzt, xyz, feat_bf, W1.T, b1.reshape(1, 64), W2.T,
      b2.reshape(1, 128))

    new_features = jnp.transpose(pooled, (0, 2, 1))
    return (new_xyz, new_features, new_curvature)


# speedup vs baseline: 1.7978x; 1.7978x over previous
import numpy as np
import jax
import jax.numpy as jnp
from jax.experimental import pallas as pl
from jax.experimental.pallas import tpu as pltpu

_N = 8192
_M = 3276
_M4 = 3328
_S = 512
_K = 32
_TQ = 128
_R2 = np.float32(0.2 * 0.2)
_BIG = np.float32(1e9)
_BIGN = np.float32(_N)


def _select_kernel(ncol_ref, nrow_ref, x4_ref, out_ref):
    n_col = ncol_ref[0]
    ii = jax.lax.broadcasted_iota(jnp.int32, (_N, 1), 0).astype(jnp.float32)
    TJ = 128
    jj0 = jax.lax.broadcasted_iota(jnp.int32, (1, TJ), 1).astype(jnp.float32)

    def rank_body(jt, rank):
        nj = nrow_ref[0, :, pl.ds(jt * TJ, TJ)]
        jj = jj0 + jnp.float32(jt) * np.float32(TJ)
        less = (nj < n_col) | ((nj == n_col) & (jj < ii))
        return rank + jnp.sum(less.astype(jnp.float32), axis=1, keepdims=True)

    rank = jax.lax.fori_loop(0, _N // TJ, rank_body,
                             jnp.zeros((_N, 1), jnp.float32))
    TR = 128
    rr0 = jax.lax.broadcasted_iota(jnp.int32, (1, TR), 1).astype(jnp.float32)

    def scatter_body(rt, _):
        rr = rr0 + jnp.float32(rt) * np.float32(TR)
        e = (rank == rr).astype(jnp.float32)
        for d in range(4):
            col = x4_ref[0, :, d:d + 1]
            out_ref[0, d:d + 1, pl.ds(rt * TR, TR)] = jnp.sum(
                e * col, axis=0, keepdims=True)
        return 0

    jax.lax.fori_loop(0, _M4 // TR, scatter_body, 0)


def _fps_kernel(xc_ref, curv_ref, ox_ref, oy_ref, oz_ref, oc_ref):
    xc = xc_ref[...]
    x0 = xc[:, 0, :]
    x1 = xc[:, 1, :]
    x2 = xc[:, 2, :]
    x3 = xc[:, 3, :]
    curv = curv_ref[...]
    Bb = x0.shape[0]
    li = jax.lax.broadcasted_iota(jnp.int32, (Bb, _M4), 1).astype(jnp.float32)
    mask = li < np.float32(_M)
    li_n = jax.lax.broadcasted_iota(jnp.int32, (Bb, _N), 1).astype(jnp.float32)

    li_s = jax.lax.broadcasted_iota(jnp.int32, (Bb, _S), 1)

    sel0 = li_s == 0
    z = jnp.zeros((Bb, _S), jnp.float32)
    a0 = jnp.where(sel0, x0[:, 0:1], z)
    a1 = jnp.where(sel0, x1[:, 0:1], z)
    a2 = jnp.where(sel0, x2[:, 0:1], z)
    ac = jnp.where(sel0, curv[:, 0:1], z)
    dists0 = jnp.full((Bb, _M4), 1e10, jnp.float32)

    def body(t, carry):
        dists, l0, l1, l2, l3, a0, a1, a2, ac = carry
        d = ((((x0 - l0) * (x0 - l0)) + ((x1 - l1) * (x1 - l1)))
             + ((x2 - l2) * (x2 - l2))) + ((x3 - l3) * (x3 - l3))
        dists = jnp.where(mask, jnp.minimum(dists, d), -1.0)
        m = jnp.max(dists, axis=1, keepdims=True)
        nxt = jnp.min(jnp.where(dists == m, li, _BIG), axis=1, keepdims=True)
        oh = (li == nxt).astype(jnp.float32)
        g0 = jnp.sum(x0 * oh, axis=1, keepdims=True)
        g1 = jnp.sum(x1 * oh, axis=1, keepdims=True)
        g2 = jnp.sum(x2 * oh, axis=1, keepdims=True)
        g3 = jnp.sum(x3 * oh, axis=1, keepdims=True)
        gc = jnp.sum(curv * (li_n == nxt).astype(jnp.float32), axis=1, keepdims=True)
        sel = li_s == t
        a0 = jnp.where(sel, g0, a0)
        a1 = jnp.where(sel, g1, a1)
        a2 = jnp.where(sel, g2, a2)
        ac = jnp.where(sel, gc, ac)
        return (dists, g0, g1, g2, g3, a0, a1, a2, ac)

    carry0 = (dists0, x0[:, 0:1], x1[:, 0:1], x2[:, 0:1], x3[:, 0:1],
              a0, a1, a2, ac)
    _, _, _, _, _, a0, a1, a2, ac = jax.lax.fori_loop(1, _S, body, carry0)
    ox_ref[...] = a0
    oy_ref[...] = a1
    oz_ref[...] = a2
    oc_ref[...] = ac


def _group_mlp_kernel(q_ref, xt_ref, xf_ref, w1_ref, b1_ref, w2_ref,
                      b2_ref, out_ref, masked_ref, first_ref, pooled_ref):
    q = q_ref[0]
    xt = xt_ref[0]
    xf = xf_ref[0]
    qn = (q[:, 0:1] * q[:, 0:1] + q[:, 1:2] * q[:, 1:2]) + q[:, 2:3] * q[:, 2:3]
    xn = (xt[0:1, :] * xt[0:1, :] + xt[1:2, :] * xt[1:2, :]) + xt[2:3, :] * xt[2:3, :]
    cross = jnp.dot(q, xt, preferred_element_type=jnp.float32)
    d2 = (qn + xn) - 2.0 * cross
    li = jax.lax.broadcasted_iota(jnp.int32, (_TQ, _N), 1).astype(jnp.float32)
    masked_ref[...] = jnp.where(d2 > _R2, _BIGN, li)
    pooled_ref[...] = jnp.zeros((_TQ, 128), jnp.float32)
    first_ref[...] = jnp.zeros((_TQ, 1), jnp.float32)

    def body(k, _):
        msk = masked_ref[...]
        cur = jnp.min(msk, axis=1, keepdims=True)
        first = jnp.where(k == 0, cur, first_ref[...])
        first_ref[...] = first
        sel = jnp.where(cur >= _BIGN, first, cur)
        sel = jnp.minimum(sel, np.float32(_N - 1))
        oh = (li == sel).astype(jnp.float32)
        masked_ref[...] = jnp.where(msk == cur, _BIGN, msk)
        g = jax.lax.dot_general(oh, xf, (((1,), (0,)), ((), ())),
                                preferred_element_type=jnp.float32)
        h = jnp.concatenate([g[:, :3] - q, g[:, 3:]], axis=1)
        l1 = jnp.maximum(jnp.dot(h, w1_ref[...], preferred_element_type=jnp.float32)
                         + b1_ref[...], 0.0)
        l2 = jnp.maximum(jnp.dot(l1, w2_ref[...], preferred_element_type=jnp.float32)
                         + b2_ref[...], 0.0)
        pooled_ref[...] = jnp.maximum(pooled_ref[...], l2)
        return 0

    jax.lax.fori_loop(0, _K, body, 0)
    out_ref[0] = pooled_ref[...]


def kernel(xyz, features, curvature, W1, b1, W2, b2):
    Bb, Nn, _ = xyz.shape
    C = features.shape[-1]

    x4 = jnp.concatenate([xyz, curvature[:, :, None]], axis=2)
    norms = jnp.linalg.norm(xyz, axis=2)

    xc = pl.pallas_call(
        _select_kernel,
        grid=(Bb,),
        in_specs=[
            pl.BlockSpec((1, _N, 1), lambda b: (b, 0, 0)),
            pl.BlockSpec((1, 1, _N), lambda b: (b, 0, 0)),
            pl.BlockSpec((1, _N, 4), lambda b: (b, 0, 0)),
        ],
        out_specs=pl.BlockSpec((1, 4, _M4), lambda b: (b, 0, 0)),
        out_shape=jax.ShapeDtypeStruct((Bb, 4, _M4), jnp.float32),
    )(norms[:, :, None], norms[:, None, :], x4)

    ox, oy, oz, oc = pl.pallas_call(
        _fps_kernel,
        out_shape=[jax.ShapeDtypeStruct((Bb, _S), jnp.float32)] * 4,
    )(xc, curvature)

    new_xyz = jnp.stack([ox, oy, oz], axis=-1)
    new_curvature = oc

    xf = jnp.concatenate([xyz, features], axis=2)
    xyzt = jnp.transpose(xyz, (0, 2, 1))

    pooled = pl.pallas_call(
        _group_mlp_kernel,
        grid=(Bb, _S // _TQ),
        in_specs=[
            pl.BlockSpec((1, _TQ, 3), lambda b, s: (b, s, 0)),
            pl.BlockSpec((1, 3, _N), lambda b, s: (b, 0, 0)),
            pl.BlockSpec((1, _N, 3 + C), lambda b, s: (b, 0, 0)),
            pl.BlockSpec(W1.T.shape, lambda b, s: (0, 0)),
            pl.BlockSpec((1, 64), lambda b, s: (0, 0)),
            pl.BlockSpec(W2.T.shape, lambda b, s: (0, 0)),
            pl.BlockSpec((1, 128), lambda b, s: (0, 0)),
        ],
        out_specs=pl.BlockSpec((1, _TQ, 128), lambda b, s: (b, s, 0)),
        out_shape=jax.ShapeDtypeStruct((Bb, _S, 128), jnp.float32),
        scratch_shapes=[
            pltpu.VMEM((_TQ, _N), jnp.float32),
            pltpu.VMEM((_TQ, 1), jnp.float32),
            pltpu.VMEM((_TQ, 128), jnp.float32),
        ],
    )(new_xyz, xyzt, xf, W1.T, b1.reshape(1, 64), W2.T, b2.reshape(1, 128))

    new_features = jnp.transpose(pooled, (0, 2, 1))
    return (new_xyz, new_features, new_curvature)


# speedup vs baseline: 2.0046x; 1.1151x over previous
import numpy as np
import jax
import jax.numpy as jnp
from jax.experimental import pallas as pl
from jax.experimental.pallas import tpu as pltpu

_N = 8192
_M = 3276
_M4 = 3328
_S = 512
_K = 32
_TQ = 128
_R2 = np.float32(0.2 * 0.2)
_BIG = np.float32(1e9)
_BIGN = np.float32(_N)


def _select_kernel(ncol_ref, nrow_ref, x4_ref, out_ref):
    n_col = ncol_ref[0]
    ii = jax.lax.broadcasted_iota(jnp.int32, (_N, 1), 0).astype(jnp.float32)
    TJ = 128
    jj0 = jax.lax.broadcasted_iota(jnp.int32, (1, TJ), 1).astype(jnp.float32)

    def rank_body(jt, rank):
        nj = nrow_ref[0, :, pl.ds(jt * TJ, TJ)]
        jj = jj0 + jnp.float32(jt) * np.float32(TJ)
        less = (nj < n_col) | ((nj == n_col) & (jj < ii))
        return rank + jnp.sum(less.astype(jnp.float32), axis=1, keepdims=True)

    rank = jax.lax.fori_loop(0, _N // TJ, rank_body,
                             jnp.zeros((_N, 1), jnp.float32))
    TR = 128
    rr0 = jax.lax.broadcasted_iota(jnp.int32, (1, TR), 1).astype(jnp.float32)

    def scatter_body(rt, _):
        rr = rr0 + jnp.float32(rt) * np.float32(TR)
        e = (rank == rr).astype(jnp.float32)
        for d in range(4):
            col = x4_ref[0, :, d:d + 1]
            out_ref[0, d:d + 1, pl.ds(rt * TR, TR)] = jnp.sum(
                e * col, axis=0, keepdims=True)
        return 0

    jax.lax.fori_loop(0, _M4 // TR, scatter_body, 0)


def _fps_kernel(xc_ref, curv_ref, ox_ref, oy_ref, oz_ref, oc_ref):
    xc = xc_ref[...]
    x0 = xc[:, 0, :]
    x1 = xc[:, 1, :]
    x2 = xc[:, 2, :]
    x3 = xc[:, 3, :]
    curv = curv_ref[...]
    Bb = x0.shape[0]
    li = jax.lax.broadcasted_iota(jnp.int32, (Bb, _M4), 1).astype(jnp.float32)
    mask = li < np.float32(_M)
    li_n = jax.lax.broadcasted_iota(jnp.int32, (Bb, _N), 1).astype(jnp.float32)

    li_s = jax.lax.broadcasted_iota(jnp.int32, (Bb, _S), 1)

    sel0 = li_s == 0
    z = jnp.zeros((Bb, _S), jnp.float32)
    a0 = jnp.where(sel0, x0[:, 0:1], z)
    a1 = jnp.where(sel0, x1[:, 0:1], z)
    a2 = jnp.where(sel0, x2[:, 0:1], z)
    ac = jnp.where(sel0, curv[:, 0:1], z)
    dists0 = jnp.full((Bb, _M4), 1e10, jnp.float32)

    def body(t, carry):
        dists, l0, l1, l2, l3, a0, a1, a2, ac = carry
        d = ((((x0 - l0) * (x0 - l0)) + ((x1 - l1) * (x1 - l1)))
             + ((x2 - l2) * (x2 - l2))) + ((x3 - l3) * (x3 - l3))
        dists = jnp.where(mask, jnp.minimum(dists, d), -1.0)
        m = jnp.max(dists, axis=1, keepdims=True)
        nxt = jnp.min(jnp.where(dists == m, li, _BIG), axis=1, keepdims=True)
        oh = (li == nxt).astype(jnp.float32)
        g0 = jnp.sum(x0 * oh, axis=1, keepdims=True)
        g1 = jnp.sum(x1 * oh, axis=1, keepdims=True)
        g2 = jnp.sum(x2 * oh, axis=1, keepdims=True)
        g3 = jnp.sum(x3 * oh, axis=1, keepdims=True)
        gc = jnp.sum(curv * (li_n == nxt).astype(jnp.float32), axis=1, keepdims=True)
        sel = li_s == t
        a0 = jnp.where(sel, g0, a0)
        a1 = jnp.where(sel, g1, a1)
        a2 = jnp.where(sel, g2, a2)
        ac = jnp.where(sel, gc, ac)
        return (dists, g0, g1, g2, g3, a0, a1, a2, ac)

    carry0 = (dists0, x0[:, 0:1], x1[:, 0:1], x2[:, 0:1], x3[:, 0:1],
              a0, a1, a2, ac)
    _, _, _, _, _, a0, a1, a2, ac = jax.lax.fori_loop(1, _S, body, carry0)
    ox_ref[...] = a0
    oy_ref[...] = a1
    oz_ref[...] = a2
    oc_ref[...] = ac


def _group_mlp_kernel(q_ref, xt_ref, xf_ref, w1_ref, b1_ref, w2_ref,
                      b2_ref, out_ref, slot_ref, pooled_ref):
    q = q_ref[0]
    xt = xt_ref[0]
    xf = xf_ref[0]
    qn = (q[:, 0:1] * q[:, 0:1] + q[:, 1:2] * q[:, 1:2]) + q[:, 2:3] * q[:, 2:3]
    xn = (xt[0:1, :] * xt[0:1, :] + xt[1:2, :] * xt[1:2, :]) + xt[2:3, :] * xt[2:3, :]
    cross = jnp.dot(q, xt, preferred_element_type=jnp.float32)
    d2 = (qn + xn) - 2.0 * cross
    inr = (d2 <= _R2).astype(jnp.float32)

    TC = 128
    ci = jax.lax.broadcasted_iota(jnp.int32, (TC, TC), 0)
    cj = jax.lax.broadcasted_iota(jnp.int32, (TC, TC), 1)
    lt = (ci < cj).astype(jnp.float32)
    off = jnp.zeros((_TQ, 1), jnp.float32)
    for c in range(_N // TC):
        ch = inr[:, c * TC:(c + 1) * TC]
        pc = jax.lax.dot_general(ch, lt, (((1,), (0,)), ((), ())),
                                 preferred_element_type=jnp.float32) + off
        slot_ref[:, c * TC:(c + 1) * TC] = jnp.where(
            (ch > 0.0) & (pc < np.float32(_K)), pc, _BIGN)
        off = off + jnp.sum(ch, axis=1, keepdims=True)
    hitcnt = off
    pad0 = hitcnt == 0.0
    li = jax.lax.broadcasted_iota(jnp.int32, (_TQ, _N), 1)
    lastcol = (li == (_N - 1)).astype(jnp.float32)
    pooled_ref[...] = jnp.zeros((_TQ, 128), jnp.float32)

    def body(k, _):
        kf = k.astype(jnp.float32)
        sel = jnp.where(kf < hitcnt, kf, 0.0)
        oh = jnp.where(pad0, lastcol,
                       (slot_ref[...] == sel).astype(jnp.float32))
        g = jax.lax.dot_general(oh, xf, (((1,), (0,)), ((), ())),
                                preferred_element_type=jnp.float32)
        h = jnp.concatenate([g[:, :3] - q, g[:, 3:]], axis=1)
        l1 = jnp.maximum(jnp.dot(h, w1_ref[...], preferred_element_type=jnp.float32)
                         + b1_ref[...], 0.0)
        l2 = jnp.maximum(jnp.dot(l1, w2_ref[...], preferred_element_type=jnp.float32)
                         + b2_ref[...], 0.0)
        pooled_ref[...] = jnp.maximum(pooled_ref[...], l2)
        return 0

    jax.lax.fori_loop(0, _K, body, 0)
    out_ref[0] = pooled_ref[...]


def kernel(xyz, features, curvature, W1, b1, W2, b2):
    Bb, Nn, _ = xyz.shape
    C = features.shape[-1]

    x4 = jnp.concatenate([xyz, curvature[:, :, None]], axis=2)
    norms = jnp.linalg.norm(xyz, axis=2)

    xc = pl.pallas_call(
        _select_kernel,
        grid=(Bb,),
        in_specs=[
            pl.BlockSpec((1, _N, 1), lambda b: (b, 0, 0)),
            pl.BlockSpec((1, 1, _N), lambda b: (b, 0, 0)),
            pl.BlockSpec((1, _N, 4), lambda b: (b, 0, 0)),
        ],
        out_specs=pl.BlockSpec((1, 4, _M4), lambda b: (b, 0, 0)),
        out_shape=jax.ShapeDtypeStruct((Bb, 4, _M4), jnp.float32),
    )(norms[:, :, None], norms[:, None, :], x4)

    ox, oy, oz, oc = pl.pallas_call(
        _fps_kernel,
        out_shape=[jax.ShapeDtypeStruct((Bb, _S), jnp.float32)] * 4,
    )(xc, curvature)

    new_xyz = jnp.stack([ox, oy, oz], axis=-1)
    new_curvature = oc

    xf = jnp.concatenate([xyz, features], axis=2)
    xyzt = jnp.transpose(xyz, (0, 2, 1))

    pooled = pl.pallas_call(
        _group_mlp_kernel,
        grid=(Bb, _S // _TQ),
        in_specs=[
            pl.BlockSpec((1, _TQ, 3), lambda b, s: (b, s, 0)),
            pl.BlockSpec((1, 3, _N), lambda b, s: (b, 0, 0)),
            pl.BlockSpec((1, _N, 3 + C), lambda b, s: (b, 0, 0)),
            pl.BlockSpec(W1.T.shape, lambda b, s: (0, 0)),
            pl.BlockSpec((1, 64), lambda b, s: (0, 0)),
            pl.BlockSpec(W2.T.shape, lambda b, s: (0, 0)),
            pl.BlockSpec((1, 128), lambda b, s: (0, 0)),
        ],
        out_specs=pl.BlockSpec((1, _TQ, 128), lambda b, s: (b, s, 0)),
        out_shape=jax.ShapeDtypeStruct((Bb, _S, 128), jnp.float32),
        scratch_shapes=[
            pltpu.VMEM((_TQ, _N), jnp.float32),
            pltpu.VMEM((_TQ, 128), jnp.float32),
        ],
    )(new_xyz, xyzt, xf, W1.T, b1.reshape(1, 64), W2.T, b2.reshape(1, 128))

    new_features = jnp.transpose(pooled, (0, 2, 1))
    return (new_xyz, new_features, new_curvature)
